# baseline (speedup 1.0000x reference)
import functools

import jax
import jax.numpy as jnp
from jax import lax
from jax.experimental import pallas as pl
from jax.experimental.pallas import tpu as pltpu
from jax.experimental.pallas import tpu_sc as plsc

_N = 10000
_M = 32
_D = 128
_NC = 2
_NS = 16
_NW = _NC * _NS
_NPAD = 10240
_R = _NPAD * _M
_RW = _R // _NW
_CH = 128
_G = _RW // _CH
_NCHUNK = 4
_NPC = _NPAD // _NCHUNK
_RC = _R // _NCHUNK
_GC = _RC // _NW // _CH
_NBLK = 256
_EPS = 1e-12


def _prep_body(x_ref, nb_ref, xn_ref, xnh_ref, idx_ref):
    xb = x_ref[...]
    nrm = jnp.sqrt(jnp.sum(xb * xb, axis=1, keepdims=True))
    xn = xb / jnp.maximum(nrm, _EPS)
    xn_ref[...] = xn
    xnh_ref[...] = xn.astype(jnp.bfloat16)
    idx_ref[...] = nb_ref[...] - 1


def _prep(x_pad, nb_pad):
    return pl.pallas_call(
        _prep_body,
        out_shape=[
            jax.ShapeDtypeStruct((_NPAD, _D), jnp.float32),
            jax.ShapeDtypeStruct((_NPAD, _D), jnp.bfloat16),
            jax.ShapeDtypeStruct((_NPAD, _M), jnp.int32),
        ],
    )(x_pad, nb_pad)


_K = 6


def _sc_gather_body(xn_hbm, idx_hbm, z_hbm, idx_v, zbuf, gsem, wsem):
    wid = lax.axis_index("s") * _NC + lax.axis_index("c")

    pltpu.sync_copy(idx_hbm.at[wid], idx_v)

    def gather(g, b):
        pltpu.async_copy(xn_hbm.at[idx_v.at[g]], zbuf.at[b], gsem)

    def gather_wait(b):
        pltpu.make_async_copy(xn_hbm.at[idx_v.at[0]], zbuf.at[b], gsem).wait()

    def write(g, b):
        pltpu.async_copy(zbuf.at[b],
                         z_hbm.at[pl.ds((wid * _GC + g) * _CH, _CH)], wsem)

    def write_wait(b):
        pltpu.make_async_copy(zbuf.at[b], z_hbm.at[pl.ds(0, _CH)], wsem).wait()

    for b in range(_K):
        gather(b, b)

    def step(g, carry):
        b = lax.rem(g, _K)
        gather_wait(b)
        write(g, b)
        prev = lax.rem(g + _K - 1, _K)

        @pl.when(g > 0)
        def _():
            write_wait(prev)

        @pl.when(jnp.logical_and(g > 0, g + _K - 1 < _GC))
        def _():
            gather(g + _K - 1, prev)
        return carry

    lax.fori_loop(0, _GC, step, 0)
    write_wait(lax.rem(_GC - 1, _K))


def _sc_gather(xn, idx2d):
    mesh = plsc.VectorSubcoreMesh(core_axis_name="c", subcore_axis_name="s")
    f = pl.kernel(
        _sc_gather_body,
        out_type=jax.ShapeDtypeStruct((_RC, _D // 2), jnp.int32),
        mesh=mesh,
        scratch_types=[
            pltpu.VMEM((_GC, _CH), jnp.int32),
            pltpu.VMEM((_K, _CH, _D // 2), jnp.int32),
            pltpu.SemaphoreType.DMA,
            pltpu.SemaphoreType.DMA,
        ],
        compiler_params=pltpu.CompilerParams(use_tc_tiling_on_sc=False),
    )
    return f(xn, idx2d)


def _route_body(z_ref, xn_ref, out_ref):
    zi = z_ref[...]
    zlo = lax.bitcast_convert_type(zi << 16, jnp.float32)
    zhi = lax.bitcast_convert_type(zi & jnp.int32(-65536), jnp.float32)
    xn = xn_ref[...]
    xlo, xhi = xn[:, :_D // 2], xn[:, _D // 2:]
    ulo = jnp.mean(zlo, axis=1) + xlo
    uhi = jnp.mean(zhi, axis=1) + xhi
    for _ in range(2):
        nrm2 = (jnp.sum(ulo * ulo, axis=1, keepdims=True)
                + jnp.sum(uhi * uhi, axis=1, keepdims=True))
        squash = nrm2 / (nrm2 + 1.0)
        sc = squash / jnp.maximum(jnp.sqrt(nrm2), _EPS)
        vlo, vhi = sc * ulo, sc * uhi
        p = (jnp.sum(zlo * vlo[:, None, :], axis=2)
             + jnp.sum(zhi * vhi[:, None, :], axis=2))
        p = jax.nn.softmax(p, axis=1)
        ulo = jnp.sum(zlo * p[:, :, None], axis=1) + xlo
        uhi = jnp.sum(zhi * p[:, :, None], axis=1) + xhi
    out_ref[...] = jnp.concatenate([ulo, uhi], axis=1)


def _route(z3, xn):
    grid = (_NPC // _NBLK,)
    return pl.pallas_call(
        _route_body,
        grid=grid,
        in_specs=[
            pl.BlockSpec((_NBLK, _M, _D // 2), lambda i: (i, 0, 0)),
            pl.BlockSpec((_NBLK, _D), lambda i: (i, 0)),
        ],
        out_specs=pl.BlockSpec((_NBLK, _D), lambda i: (i, 0)),
        out_shape=jax.ShapeDtypeStruct((_NPC, _D), jnp.float32),
        compiler_params=pltpu.CompilerParams(
            dimension_semantics=("arbitrary",)),
    )(z3, xn)


def kernel(x, x_nb):
    n, d = x.shape
    x_pad = jnp.pad(x, ((0, _NPAD - n), (0, 0)))
    nb_pad = jnp.pad(x_nb, ((0, _NPAD - n), (0, 0)), constant_values=1)
    xn, xnh, idx = _prep(x_pad, nb_pad)
    lo = lax.bitcast_convert_type(xnh[:, :_D // 2],
                                  jnp.uint16).astype(jnp.uint32)
    hi = lax.bitcast_convert_type(xnh[:, _D // 2:],
                                  jnp.uint16).astype(jnp.uint32)
    xnp = lax.bitcast_convert_type(lo | (hi << 16), jnp.int32)
    idx2d = idx.reshape(_R // _CH, _CH)
    us = []
    for c in range(_NCHUNK):
        rows = idx2d[c * (_RC // _CH):(c + 1) * (_RC // _CH)].reshape(
            _NW, _GC, _CH)
        z = _sc_gather(xnp, rows)
        us.append(_route(z.reshape(_NPC, _M, _D // 2),
                         xn[c * _NPC:(c + 1) * _NPC]))
    u = jnp.concatenate(us, axis=0)
    return u[:n]


# speedup vs baseline: 1.3380x; 1.3380x over previous
import functools

import jax
import jax.numpy as jnp
from jax import lax
from jax.experimental import pallas as pl
from jax.experimental.pallas import tpu as pltpu
from jax.experimental.pallas import tpu_sc as plsc

_N = 10000
_M = 32
_D = 128
_NC = 2
_NS = 16
_NW = _NC * _NS
_NPAD = 10240
_R = _NPAD * _M
_RW = _R // _NW
_CH = 128
_G = _RW // _CH
_NCHUNK = 8
_NPC = _NPAD // _NCHUNK
_RC = _R // _NCHUNK
_GC = _RC // _NW // _CH
_NBLK = 256
_EPS = 1e-12


def _prep_body(x_ref, nb_ref, xn_ref, xnh_ref, idx_ref):
    xb = x_ref[...]
    nrm = jnp.sqrt(jnp.sum(xb * xb, axis=1, keepdims=True))
    xn = xb / jnp.maximum(nrm, _EPS)
    xn_ref[...] = xn
    xnh_ref[...] = xn.astype(jnp.bfloat16)
    idx_ref[...] = nb_ref[...] - 1


def _prep(x_pad, nb_pad):
    return pl.pallas_call(
        _prep_body,
        out_shape=[
            jax.ShapeDtypeStruct((_NPAD, _D), jnp.float32),
            jax.ShapeDtypeStruct((_NPAD, _D), jnp.bfloat16),
            jax.ShapeDtypeStruct((_NPAD, _M), jnp.int32),
        ],
    )(x_pad, nb_pad)


_K = 6


def _sc_gather_body(xn_hbm, idx_hbm, z_hbm, idx_v, zbuf, gsem, wsem):
    wid = lax.axis_index("s") * _NC + lax.axis_index("c")

    pltpu.sync_copy(idx_hbm.at[wid], idx_v)

    def gather(g, b):
        pltpu.async_copy(xn_hbm.at[idx_v.at[g]], zbuf.at[b], gsem)

    def gather_wait(b):
        pltpu.make_async_copy(xn_hbm.at[idx_v.at[0]], zbuf.at[b], gsem).wait()

    def write(g, b):
        pltpu.async_copy(zbuf.at[b],
                         z_hbm.at[pl.ds((wid * _GC + g) * _CH, _CH)], wsem)

    def write_wait(b):
        pltpu.make_async_copy(zbuf.at[b], z_hbm.at[pl.ds(0, _CH)], wsem).wait()

    for b in range(_K):
        gather(b, b)

    def step(g, carry):
        b = lax.rem(g, _K)
        gather_wait(b)
        write(g, b)
        prev = lax.rem(g + _K - 1, _K)

        @pl.when(g > 0)
        def _():
            write_wait(prev)

        @pl.when(jnp.logical_and(g > 0, g + _K - 1 < _GC))
        def _():
            gather(g + _K - 1, prev)
        return carry

    lax.fori_loop(0, _GC, step, 0)
    write_wait(lax.rem(_GC - 1, _K))


def _sc_gather(xn, idx2d):
    mesh = plsc.VectorSubcoreMesh(core_axis_name="c", subcore_axis_name="s")
    f = pl.kernel(
        _sc_gather_body,
        out_type=jax.ShapeDtypeStruct((_RC, _D), jnp.float32),
        mesh=mesh,
        scratch_types=[
            pltpu.VMEM((_GC, _CH), jnp.int32),
            pltpu.VMEM((_K, _CH, _D), jnp.float32),
            pltpu.SemaphoreType.DMA,
            pltpu.SemaphoreType.DMA,
        ],
        compiler_params=pltpu.CompilerParams(use_tc_tiling_on_sc=False),
    )
    return f(xn, idx2d)


def _route_body(z_ref, xn_ref, out_ref):
    z = z_ref[...].astype(jnp.float32)
    xn = xn_ref[...]
    u = jnp.mean(z, axis=1) + xn
    for _ in range(2):
        nrm2 = jnp.sum(u * u, axis=1, keepdims=True)
        squash = nrm2 / (nrm2 + 1.0)
        v = squash * u / jnp.maximum(jnp.sqrt(nrm2), _EPS)
        p = jnp.sum(z * v[:, None, :], axis=2)
        p = jax.nn.softmax(p, axis=1)
        u = jnp.sum(z * p[:, :, None], axis=1) + xn
    out_ref[...] = u


def _route(z3, xn):
    grid = (_NPC // _NBLK,)
    return pl.pallas_call(
        _route_body,
        grid=grid,
        in_specs=[
            pl.BlockSpec((_NBLK, _M, _D), lambda i: (i, 0, 0)),
            pl.BlockSpec((_NBLK, _D), lambda i: (i, 0)),
        ],
        out_specs=pl.BlockSpec((_NBLK, _D), lambda i: (i, 0)),
        out_shape=jax.ShapeDtypeStruct((_NPC, _D), jnp.float32),
        compiler_params=pltpu.CompilerParams(
            dimension_semantics=("arbitrary",)),
    )(z3, xn)


def kernel(x, x_nb):
    n, d = x.shape
    x_pad = jnp.pad(x, ((0, _NPAD - n), (0, 0)))
    nb_pad = jnp.pad(x_nb, ((0, _NPAD - n), (0, 0)), constant_values=1)
    xn, _xnh, idx = _prep(x_pad, nb_pad)
    idx2d = idx.reshape(_R // _CH, _CH)
    us = []
    for c in range(_NCHUNK):
        rows = idx2d[c * (_RC // _CH):(c + 1) * (_RC // _CH)].reshape(
            _NW, _GC, _CH)
        z = _sc_gather(xn, rows)
        us.append(_route(z.reshape(_NPC, _M, _D),
                         xn[c * _NPC:(c + 1) * _NPC]))
    u = jnp.concatenate(us, axis=0)
    return u[:n]


# speedup vs baseline: 1.4205x; 1.0616x over previous
import functools

import jax
import jax.numpy as jnp
from jax import lax
from jax.experimental import pallas as pl
from jax.experimental.pallas import tpu as pltpu
from jax.experimental.pallas import tpu_sc as plsc

_N = 10000
_M = 32
_D = 128
_NC = 2
_NS = 16
_NW = _NC * _NS
_NPAD = 10240
_R = _NPAD * _M
_RW = _R // _NW
_CH = 128
_G = _RW // _CH
_NCHUNK = 4
_NPC = _NPAD // _NCHUNK
_RC = _R // _NCHUNK
_GC = _RC // _NW // _CH
_NBLK = 256
_EPS = 1e-12


def _prep_body(x_ref, nb_ref, xn_ref, xnh_ref, idx_ref):
    xb = x_ref[...]
    nrm = jnp.sqrt(jnp.sum(xb * xb, axis=1, keepdims=True))
    xn = xb / jnp.maximum(nrm, _EPS)
    xn_ref[...] = xn
    xnh_ref[...] = xn.astype(jnp.bfloat16)
    idx_ref[...] = nb_ref[...] - 1


def _prep(x_pad, nb_pad):
    return pl.pallas_call(
        _prep_body,
        out_shape=[
            jax.ShapeDtypeStruct((_NPAD, _D), jnp.float32),
            jax.ShapeDtypeStruct((_NPAD, _D), jnp.bfloat16),
            jax.ShapeDtypeStruct((_NPAD, _M), jnp.int32),
        ],
    )(x_pad, nb_pad)


_K = 6


def _sc_gather_body(xn_hbm, idx_hbm, z_hbm, idx_v, zbuf, gsem, wsem):
    wid = lax.axis_index("s") * _NC + lax.axis_index("c")

    pltpu.sync_copy(idx_hbm.at[wid], idx_v)

    def gather(g, b):
        pltpu.async_copy(xn_hbm.at[idx_v.at[g]], zbuf.at[b], gsem)

    def gather_wait(b):
        pltpu.make_async_copy(xn_hbm.at[idx_v.at[0]], zbuf.at[b], gsem).wait()

    def write(g, b):
        pltpu.async_copy(zbuf.at[b],
                         z_hbm.at[pl.ds((wid * _GC + g) * _CH, _CH)], wsem)

    def write_wait(b):
        pltpu.make_async_copy(zbuf.at[b], z_hbm.at[pl.ds(0, _CH)], wsem).wait()

    for b in range(_K):
        gather(b, b)

    def step(g, carry):
        b = lax.rem(g, _K)
        gather_wait(b)
        write(g, b)
        prev = lax.rem(g + _K - 1, _K)

        @pl.when(g > 0)
        def _():
            write_wait(prev)

        @pl.when(jnp.logical_and(g > 0, g + _K - 1 < _GC))
        def _():
            gather(g + _K - 1, prev)
        return carry

    lax.fori_loop(0, _GC, step, 0)
    write_wait(lax.rem(_GC - 1, _K))


def _sc_gather(xn, idx2d):
    mesh = plsc.VectorSubcoreMesh(core_axis_name="c", subcore_axis_name="s")
    f = pl.kernel(
        _sc_gather_body,
        out_type=jax.ShapeDtypeStruct((_RC, _D), jnp.float32),
        mesh=mesh,
        scratch_types=[
            pltpu.VMEM((_GC, _CH), jnp.int32),
            pltpu.VMEM((_K, _CH, _D), jnp.float32),
            pltpu.SemaphoreType.DMA,
            pltpu.SemaphoreType.DMA,
        ],
        compiler_params=pltpu.CompilerParams(use_tc_tiling_on_sc=False),
    )
    return f(xn, idx2d)


def _route_body(z_ref, xn_ref, out_ref):
    z = z_ref[...].astype(jnp.float32)
    xn = xn_ref[...]
    u = jnp.mean(z, axis=1) + xn
    for _ in range(2):
        nrm2 = jnp.sum(u * u, axis=1, keepdims=True)
        squash = nrm2 / (nrm2 + 1.0)
        v = squash * u / jnp.maximum(jnp.sqrt(nrm2), _EPS)
        p = jnp.sum(z * v[:, None, :], axis=2)
        p = jax.nn.softmax(p, axis=1)
        u = jnp.sum(z * p[:, :, None], axis=1) + xn
    out_ref[...] = u


def _route(z3, xn):
    grid = (_NPC // _NBLK,)
    return pl.pallas_call(
        _route_body,
        grid=grid,
        in_specs=[
            pl.BlockSpec((_NBLK, _M, _D), lambda i: (i, 0, 0)),
            pl.BlockSpec((_NBLK, _D), lambda i: (i, 0)),
        ],
        out_specs=pl.BlockSpec((_NBLK, _D), lambda i: (i, 0)),
        out_shape=jax.ShapeDtypeStruct((_NPC, _D), jnp.float32),
        compiler_params=pltpu.CompilerParams(
            dimension_semantics=("arbitrary",)),
    )(z3, xn)


def kernel(x, x_nb):
    n, d = x.shape
    x_pad = jnp.pad(x, ((0, _NPAD - n), (0, 0)))
    nb_pad = jnp.pad(x_nb, ((0, _NPAD - n), (0, 0)), constant_values=1)
    xn, _xnh, idx = _prep(x_pad, nb_pad)
    idx2d = idx.reshape(_R // _CH, _CH)
    us = []
    for c in range(_NCHUNK):
        rows = idx2d[c * (_RC // _CH):(c + 1) * (_RC // _CH)].reshape(
            _NW, _GC, _CH)
        z = _sc_gather(xn, rows)
        us.append(_route(z.reshape(_NPC, _M, _D),
                         xn[c * _NPC:(c + 1) * _NPC]))
    u = jnp.concatenate(us, axis=0)
    return u[:n]


# speedup vs baseline: 1.4218x; 1.0009x over previous
import jax
import jax.numpy as jnp
from jax import lax
from jax.experimental import pallas as pl
from jax.experimental.pallas import tpu as pltpu
from jax.experimental.pallas import tpu_sc as plsc

_N = 10000
_M = 32
_D = 128
_NC = 2
_NS = 16
_NW = _NC * _NS
_NPAD = 10240
_R = _NPAD * _M
_RW = _R // _NW
_CH = 128
_G = _RW // _CH
_NCHUNK = 4
_NPC = _NPAD // _NCHUNK
_RC = _R // _NCHUNK
_GC = _RC // _NW // _CH
_NBLK = 256
_EPS = 1e-12


def _prep_body(x_ref, nb_ref, xn_ref, idx_ref):
    xb = x_ref[...]
    nrm = jnp.sqrt(jnp.sum(xb * xb, axis=1, keepdims=True))
    xn_ref[...] = xb / jnp.maximum(nrm, _EPS)
    idx_ref[...] = nb_ref[...] - 1


def _prep(x_pad, nb_pad):
    return pl.pallas_call(
        _prep_body,
        out_shape=[
            jax.ShapeDtypeStruct((_NPAD, _D), jnp.float32),
            jax.ShapeDtypeStruct((_NPAD, _M), jnp.int32),
        ],
    )(x_pad, nb_pad)


_K = 6


def _sc_gather_body(xn_hbm, idx_hbm, z_hbm, idx_v, zbuf, gsem, wsem):
    wid = lax.axis_index("s") * _NC + lax.axis_index("c")

    pltpu.sync_copy(idx_hbm.at[wid], idx_v)

    def gather(g, b):
        pltpu.async_copy(xn_hbm.at[idx_v.at[g]], zbuf.at[b], gsem)

    def gather_wait(b):
        pltpu.make_async_copy(xn_hbm.at[idx_v.at[0]], zbuf.at[b], gsem).wait()

    def write(g, b):
        pltpu.async_copy(zbuf.at[b],
                         z_hbm.at[pl.ds((wid * _GC + g) * _CH, _CH)], wsem)

    def write_wait(b):
        pltpu.make_async_copy(zbuf.at[b], z_hbm.at[pl.ds(0, _CH)], wsem).wait()

    for b in range(_K):
        gather(b, b)

    def step(g, carry):
        b = lax.rem(g, _K)
        gather_wait(b)
        write(g, b)
        prev = lax.rem(g + _K - 1, _K)

        @pl.when(g > 0)
        def _():
            write_wait(prev)

        @pl.when(jnp.logical_and(g > 0, g + _K - 1 < _GC))
        def _():
            gather(g + _K - 1, prev)
        return carry

    lax.fori_loop(0, _GC, step, 0)
    write_wait(lax.rem(_GC - 1, _K))


def _sc_gather(xn, idx2d):
    mesh = plsc.VectorSubcoreMesh(core_axis_name="c", subcore_axis_name="s")
    f = pl.kernel(
        _sc_gather_body,
        out_type=jax.ShapeDtypeStruct((_RC, _D), jnp.float32),
        mesh=mesh,
        scratch_types=[
            pltpu.VMEM((_GC, _CH), jnp.int32),
            pltpu.VMEM((_K, _CH, _D), jnp.float32),
            pltpu.SemaphoreType.DMA,
            pltpu.SemaphoreType.DMA,
        ],
        compiler_params=pltpu.CompilerParams(use_tc_tiling_on_sc=False),
    )
    return f(xn, idx2d)


def _route_body(z_ref, xn_ref, out_ref):
    z = z_ref[...].astype(jnp.float32)
    xn = xn_ref[...]
    u = jnp.mean(z, axis=1) + xn
    for _ in range(2):
        nrm2 = jnp.sum(u * u, axis=1, keepdims=True)
        squash = nrm2 / (nrm2 + 1.0)
        v = squash * u / jnp.maximum(jnp.sqrt(nrm2), _EPS)
        p = jnp.sum(z * v[:, None, :], axis=2)
        p = jax.nn.softmax(p, axis=1)
        u = jnp.sum(z * p[:, :, None], axis=1) + xn
    out_ref[...] = u


def _route(z3, xn):
    grid = (_NPC // _NBLK,)
    return pl.pallas_call(
        _route_body,
        grid=grid,
        in_specs=[
            pl.BlockSpec((_NBLK, _M, _D), lambda i: (i, 0, 0)),
            pl.BlockSpec((_NBLK, _D), lambda i: (i, 0)),
        ],
        out_specs=pl.BlockSpec((_NBLK, _D), lambda i: (i, 0)),
        out_shape=jax.ShapeDtypeStruct((_NPC, _D), jnp.float32),
        compiler_params=pltpu.CompilerParams(
            dimension_semantics=("arbitrary",)),
    )(z3, xn)


def kernel(x, x_nb):
    n, d = x.shape
    x_pad = jnp.pad(x, ((0, _NPAD - n), (0, 0)))
    nb_pad = jnp.pad(x_nb, ((0, _NPAD - n), (0, 0)), constant_values=1)
    xn, idx = _prep(x_pad, nb_pad)
    idx2d = idx.reshape(_R // _CH, _CH)
    us = []
    for c in range(_NCHUNK):
        rows = idx2d[c * (_RC // _CH):(c + 1) * (_RC // _CH)].reshape(
            _NW, _GC, _CH)
        z = _sc_gather(xn, rows)
        us.append(_route(z.reshape(_NPC, _M, _D),
                         xn[c * _NPC:(c + 1) * _NPC]))
    u = jnp.concatenate(us, axis=0)
    return u[:n]
